# Initial kernel scaffold; baseline (speedup 1.0000x reference)
import jax
import jax.numpy as jnp
from jax.experimental import pallas as pl


def kernel(frame_features, segment_ids, num_segments):
    raise NotImplementedError("write your pallas kernel here")


# speedup vs baseline: 4.4974x; 4.4974x over previous
import functools

import jax
import jax.numpy as jnp
from jax import lax
from jax.experimental import pallas as pl
from jax.experimental.pallas import tpu as pltpu
from jax.experimental.pallas import tpu_sc as plsc

N = 320000
D = 128
S = 10000
NC = 2
NS = 16
NW = NC * NS
FW = N // NW
F = 80
NCHUNK = FW // F
RPT = 632
CW = 16


def _sc_body(frames_hbm, ids_hbm, zsum_hbm, zcnt_hbm, ones_hbm,
             sums_out, cnts_out,
             fbuf, idbuf, ones_v, ssum, scnt):
    cid = lax.axis_index("c")
    sid = lax.axis_index("s")
    wid = cid * NS + sid
    r0 = jnp.minimum(sid * RPT, S - RPT)

    pltpu.sync_copy(zsum_hbm.at[pl.ds(r0, RPT)], ssum.at[pl.ds(r0, RPT)])
    pltpu.sync_copy(zcnt_hbm.at[pl.ds(r0, RPT)], scnt.at[pl.ds(r0, RPT)])
    pltpu.sync_copy(ones_hbm, ones_v)
    plsc.subcore_barrier()

    base = wid * FW

    def body(g, carry):
        off = base + g * F
        pltpu.sync_copy(frames_hbm.at[pl.ds(off, F)], fbuf)
        pltpu.sync_copy(ids_hbm.at[pl.ds(off, F)], idbuf.at[0])
        pltpu.sync_copy(fbuf, ssum.at[idbuf.at[0]], add=True)
        pltpu.sync_copy(ones_v, scnt.at[idbuf.at[0]], add=True)
        return carry

    lax.fori_loop(0, NCHUNK, body, 0)
    plsc.subcore_barrier()

    pltpu.sync_copy(ssum.at[pl.ds(r0, RPT)], sums_out.at[pl.ds(cid * S + r0, RPT)])
    pltpu.sync_copy(scnt.at[pl.ds(r0, RPT)], cnts_out.at[pl.ds(cid * S + r0, RPT)])


_sc_segment_sum = functools.partial(
    pl.kernel,
    out_type=[
        jax.ShapeDtypeStruct((NC * S, D), jnp.float32),
        jax.ShapeDtypeStruct((NC * S, CW), jnp.float32),
    ],
    mesh=plsc.VectorSubcoreMesh(core_axis_name="c", subcore_axis_name="s"),
    compiler_params=pltpu.CompilerParams(use_tc_tiling_on_sc=False),
    scratch_types=[
        pltpu.VMEM((F, D), jnp.float32),
        pltpu.VMEM((1, F), jnp.int32),
        pltpu.VMEM((F, CW), jnp.float32),
        pltpu.VMEM_SHARED((S, D), jnp.float32),
        pltpu.VMEM_SHARED((S, CW), jnp.float32),
    ],
)(_sc_body)


_BS = 1000


def _combine_body(s_ref, c_ref, o_ref):
    s = s_ref[0] + s_ref[1]
    c = c_ref[0, :, 0:1] + c_ref[1, :, 0:1]
    o_ref[...] = s / (c + 1e-8)


_combine = pl.pallas_call(
    _combine_body,
    grid=(S // _BS,),
    in_specs=[
        pl.BlockSpec((2, _BS, D), lambda i: (0, i, 0)),
        pl.BlockSpec((2, _BS, CW), lambda i: (0, i, 0)),
    ],
    out_specs=pl.BlockSpec((_BS, D), lambda i: (i, 0)),
    out_shape=jax.ShapeDtypeStruct((S, D), jnp.float32),
)


def kernel(frame_features, segment_ids, num_segments):
    ids = jnp.minimum(segment_ids, num_segments - 1).astype(jnp.int32)
    zsum = jnp.zeros((S, D), jnp.float32)
    zcnt = jnp.zeros((S, CW), jnp.float32)
    ones = jnp.ones((F, CW), jnp.float32)
    sums, cnts = _sc_segment_sum(frame_features, ids, zsum, zcnt, ones)
    return _combine(sums.reshape(NC, S, D), cnts.reshape(NC, S, CW))


# speedup vs baseline: 8.9565x; 1.9915x over previous
import functools

import jax
import jax.numpy as jnp
from jax import lax
from jax.experimental import pallas as pl
from jax.experimental.pallas import tpu as pltpu
from jax.experimental.pallas import tpu_sc as plsc

N = 320000
D = 128
S = 10000
NC = 2
NS = 16
NW = NC * NS
FW = N // NW
F = 80
NCHUNK = FW // F
NBUF = 3
RPT = 632
CW = 16


def _sc_body(frames_hbm, ids_hbm, zsum_hbm, zcnt_hbm, ones_hbm,
             sums_out, cnts_out,
             fbuf, idbuf, ones_v, ssum, scnt, gsem, ssem):
    cid = lax.axis_index("c")
    sid = lax.axis_index("s")
    wid = cid * NS + sid
    r0 = jnp.minimum(sid * RPT, S - RPT)

    pltpu.sync_copy(zsum_hbm.at[pl.ds(r0, RPT)], ssum.at[pl.ds(r0, RPT)])
    pltpu.sync_copy(zcnt_hbm.at[pl.ds(r0, RPT)], scnt.at[pl.ds(r0, RPT)])
    pltpu.sync_copy(ones_hbm, ones_v)
    plsc.subcore_barrier()

    fbase = wid * FW
    ibase = wid * NCHUNK

    def issue_gather(k, bb):
        pltpu.async_copy(frames_hbm.at[pl.ds(fbase + k * F, F)],
                         fbuf.at[pl.ds(bb * F, F)], gsem.at[bb])
        pltpu.async_copy(ids_hbm.at[pl.ds(ibase + k, 1)],
                         idbuf.at[pl.ds(bb, 1)], gsem.at[bb])

    def wait_gather(bb):
        pltpu.make_async_copy(frames_hbm.at[pl.ds(0, F)],
                              fbuf.at[pl.ds(bb * F, F)], gsem.at[bb]).wait()
        pltpu.make_async_copy(ids_hbm.at[pl.ds(0, 1)],
                              idbuf.at[pl.ds(bb, 1)], gsem.at[bb]).wait()

    def issue_scatters(bb):
        row = idbuf.at[bb]
        pltpu.async_copy(fbuf.at[pl.ds(bb * F, F)], ssum.at[row],
                         ssem.at[bb], add=True)
        pltpu.async_copy(ones_v, scnt.at[row], ssem.at[bb], add=True)

    def wait_scatters(bb):
        row = idbuf.at[bb]
        pltpu.make_async_copy(fbuf.at[pl.ds(bb * F, F)], ssum.at[row],
                              ssem.at[bb]).wait()
        pltpu.make_async_copy(ones_v, scnt.at[row], ssem.at[bb]).wait()

    issue_gather(0, 0)
    issue_gather(1, 1)

    def body(k, carry):
        bb = lax.rem(k, NBUF)
        nb = lax.rem(k + 2, NBUF)
        wait_gather(bb)

        @pl.when(k >= 1)
        def _():
            wait_scatters(nb)

        @pl.when(k + 2 < NCHUNK)
        def _():
            issue_gather(k + 2, nb)

        issue_scatters(bb)
        return carry

    lax.fori_loop(0, NCHUNK, body, 0)
    wait_scatters((NCHUNK - 1) % NBUF)
    plsc.subcore_barrier()

    pltpu.sync_copy(ssum.at[pl.ds(r0, RPT)], sums_out.at[pl.ds(cid * S + r0, RPT)])
    pltpu.sync_copy(scnt.at[pl.ds(r0, RPT)], cnts_out.at[pl.ds(cid * S + r0, RPT)])


_sc_segment_sum = functools.partial(
    pl.kernel,
    out_type=[
        jax.ShapeDtypeStruct((NC * S, D), jnp.float32),
        jax.ShapeDtypeStruct((NC * S, CW), jnp.float32),
    ],
    mesh=plsc.VectorSubcoreMesh(core_axis_name="c", subcore_axis_name="s"),
    compiler_params=pltpu.CompilerParams(use_tc_tiling_on_sc=False),
    scratch_types=[
        pltpu.VMEM((NBUF * F, D), jnp.float32),
        pltpu.VMEM((NBUF, F), jnp.int32),
        pltpu.VMEM((F, CW), jnp.float32),
        pltpu.VMEM_SHARED((S, D), jnp.float32),
        pltpu.VMEM_SHARED((S, CW), jnp.float32),
        pltpu.SemaphoreType.DMA((NBUF,)),
        pltpu.SemaphoreType.DMA((NBUF,)),
    ],
)(_sc_body)


_BS = 1000


def _combine_body(s_ref, c_ref, o_ref):
    s = s_ref[0] + s_ref[1]
    c = c_ref[0, :, 0:1] + c_ref[1, :, 0:1]
    o_ref[...] = s / (c + 1e-8)


_combine = pl.pallas_call(
    _combine_body,
    grid=(S // _BS,),
    in_specs=[
        pl.BlockSpec((2, _BS, D), lambda i: (0, i, 0)),
        pl.BlockSpec((2, _BS, CW), lambda i: (0, i, 0)),
    ],
    out_specs=pl.BlockSpec((_BS, D), lambda i: (i, 0)),
    out_shape=jax.ShapeDtypeStruct((S, D), jnp.float32),
)


def kernel(frame_features, segment_ids, num_segments):
    ids = jnp.minimum(segment_ids, num_segments - 1).astype(jnp.int32)
    ids2d = ids.reshape(N // F, F)
    zsum = jnp.zeros((S, D), jnp.float32)
    zcnt = jnp.zeros((S, CW), jnp.float32)
    ones = jnp.ones((F, CW), jnp.float32)
    sums, cnts = _sc_segment_sum(frame_features, ids2d, zsum, zcnt, ones)
    return _combine(sums.reshape(NC, S, D), cnts.reshape(NC, S, CW))


# speedup vs baseline: 9.1566x; 1.0223x over previous
import functools

import jax
import jax.numpy as jnp
import numpy as np
from jax import lax
from jax.experimental import pallas as pl
from jax.experimental.pallas import tpu as pltpu
from jax.experimental.pallas import tpu_sc as plsc

N = 320000
D = 128
S = 10000
NC = 2
NS = 16
NW = NC * NS
FW = N // NW
F = 80
NCHUNK = FW // F
NBUF = 3
RPT = 632
CW = 16


def _sc_body(frames_hbm, ids_hbm, zsum_hbm, zcnt_hbm, ones_hbm,
             sums_out, cnts_out,
             fbuf, idbuf, ones_v, ssum, scnt, gsem, ssem):
    cid = lax.axis_index("c")
    sid = lax.axis_index("s")
    wid = cid * NS + sid
    r0 = jnp.minimum(sid * RPT, S - RPT)

    pltpu.sync_copy(zsum_hbm.at[pl.ds(r0, RPT)], ssum.at[pl.ds(r0, RPT)])
    pltpu.sync_copy(zcnt_hbm.at[pl.ds(r0, RPT)], scnt.at[pl.ds(r0, RPT)])
    pltpu.sync_copy(ones_hbm, ones_v)
    plsc.subcore_barrier()

    fbase = wid * FW
    ibase = wid * NCHUNK

    def issue_gather(k, bb):
        pltpu.async_copy(frames_hbm.at[pl.ds(fbase + k * F, F)],
                         fbuf.at[pl.ds(bb * F, F)], gsem.at[bb])
        pltpu.async_copy(ids_hbm.at[pl.ds(ibase + k, 1)],
                         idbuf.at[pl.ds(bb, 1)], gsem.at[bb])

    def wait_gather(bb):
        pltpu.make_async_copy(frames_hbm.at[pl.ds(0, F)],
                              fbuf.at[pl.ds(bb * F, F)], gsem.at[bb]).wait()
        pltpu.make_async_copy(ids_hbm.at[pl.ds(0, 1)],
                              idbuf.at[pl.ds(bb, 1)], gsem.at[bb]).wait()

    def issue_scatters(bb):
        row = idbuf.at[bb]
        pltpu.async_copy(fbuf.at[pl.ds(bb * F, F)], ssum.at[row],
                         ssem.at[bb], add=True)
        pltpu.async_copy(ones_v, scnt.at[row], ssem.at[bb], add=True)

    def wait_scatters(bb):
        row = idbuf.at[bb]
        pltpu.make_async_copy(fbuf.at[pl.ds(bb * F, F)], ssum.at[row],
                              ssem.at[bb]).wait()
        pltpu.make_async_copy(ones_v, scnt.at[row], ssem.at[bb]).wait()

    issue_gather(0, 0)
    issue_gather(1, 1)

    def body(k, carry):
        bb = lax.rem(k, NBUF)
        nb = lax.rem(k + 2, NBUF)
        wait_gather(bb)

        @pl.when(k >= 1)
        def _():
            wait_scatters(nb)

        @pl.when(k + 2 < NCHUNK)
        def _():
            issue_gather(k + 2, nb)

        issue_scatters(bb)
        return carry

    lax.fori_loop(0, NCHUNK, body, 0)
    wait_scatters((NCHUNK - 1) % NBUF)
    plsc.subcore_barrier()

    pltpu.sync_copy(ssum.at[pl.ds(r0, RPT)], sums_out.at[pl.ds(cid * S + r0, RPT)])
    pltpu.sync_copy(scnt.at[pl.ds(r0, RPT)], cnts_out.at[pl.ds(cid * S + r0, RPT)])


_sc_segment_sum = functools.partial(
    pl.kernel,
    out_type=[
        jax.ShapeDtypeStruct((NC * S, D), jnp.float32),
        jax.ShapeDtypeStruct((NC * S, CW), jnp.float32),
    ],
    mesh=plsc.VectorSubcoreMesh(core_axis_name="c", subcore_axis_name="s"),
    compiler_params=pltpu.CompilerParams(use_tc_tiling_on_sc=False),
    scratch_types=[
        pltpu.VMEM((NBUF * F, D), jnp.float32),
        pltpu.VMEM((NBUF, F), jnp.int32),
        pltpu.VMEM((F, CW), jnp.float32),
        pltpu.VMEM_SHARED((S, D), jnp.float32),
        pltpu.VMEM_SHARED((S, CW), jnp.float32),
        pltpu.SemaphoreType.DMA((NBUF,)),
        pltpu.SemaphoreType.DMA((NBUF,)),
    ],
)(_sc_body)


_BS = 1000


def _combine_body(s_ref, c_ref, o_ref):
    s = s_ref[0] + s_ref[1]
    c = c_ref[0, :, 0:1] + c_ref[1, :, 0:1]
    o_ref[...] = s / (c + 1e-8)


_combine = pl.pallas_call(
    _combine_body,
    grid=(S // _BS,),
    in_specs=[
        pl.BlockSpec((2, _BS, D), lambda i: (0, i, 0)),
        pl.BlockSpec((2, _BS, CW), lambda i: (0, i, 0)),
    ],
    out_specs=pl.BlockSpec((_BS, D), lambda i: (i, 0)),
    out_shape=jax.ShapeDtypeStruct((S, D), jnp.float32),
)


_ZSUM = np.zeros((S, D), np.float32)
_ZCNT = np.zeros((S, CW), np.float32)
_ONES = np.ones((F, CW), np.float32)


def kernel(frame_features, segment_ids, num_segments):
    ids2d = segment_ids.astype(jnp.int32).reshape(N // F, F)
    sums, cnts = _sc_segment_sum(frame_features, ids2d, _ZSUM, _ZCNT, _ONES)
    return _combine(sums.reshape(NC, S, D), cnts.reshape(NC, S, CW))


# speedup vs baseline: 9.2602x; 1.0113x over previous
import functools

import jax
import jax.numpy as jnp
import numpy as np
from jax import lax
from jax.experimental import pallas as pl
from jax.experimental.pallas import tpu as pltpu
from jax.experimental.pallas import tpu_sc as plsc

N = 320000
D = 128
S = 10000
NC = 2
NS = 16
NW = NC * NS
FW = N // NW
F = 80
NCHUNK = FW // F
NBUF = 3
RPT = 632
CW = 16


def _sc_body(frames_hbm, ids_hbm, zsum_hbm, zcnt_hbm, ones_hbm,
             sums_out, cnts_out,
             fbuf, idbuf, ones_v, ssum, scnt, gsem, ssem):
    cid = lax.axis_index("c")
    sid = lax.axis_index("s")
    wid = cid * NS + sid
    r0 = jnp.minimum(sid * RPT, S - RPT)

    fbase = wid * FW
    ibase = wid * NCHUNK

    def issue_gather(k, bb):
        pltpu.async_copy(frames_hbm.at[pl.ds(fbase + k * F, F)],
                         fbuf.at[pl.ds(bb * F, F)], gsem.at[bb])
        pltpu.async_copy(ids_hbm.at[pl.ds(ibase + k, 1)],
                         idbuf.at[pl.ds(bb, 1)], gsem.at[bb])

    def wait_gather(bb):
        pltpu.make_async_copy(frames_hbm.at[pl.ds(0, F)],
                              fbuf.at[pl.ds(bb * F, F)], gsem.at[bb]).wait()
        pltpu.make_async_copy(ids_hbm.at[pl.ds(0, 1)],
                              idbuf.at[pl.ds(bb, 1)], gsem.at[bb]).wait()

    def issue_scatters(bb):
        row = idbuf.at[bb]
        pltpu.async_copy(fbuf.at[pl.ds(bb * F, F)], ssum.at[row],
                         ssem.at[bb], add=True)
        pltpu.async_copy(ones_v, scnt.at[row], ssem.at[bb], add=True)

    def wait_scatters(bb):
        row = idbuf.at[bb]
        pltpu.make_async_copy(fbuf.at[pl.ds(bb * F, F)], ssum.at[row],
                              ssem.at[bb]).wait()
        pltpu.make_async_copy(ones_v, scnt.at[row], ssem.at[bb]).wait()

    issue_gather(0, 0)
    issue_gather(1, 1)

    pltpu.sync_copy(zsum_hbm.at[pl.ds(r0, RPT)], ssum.at[pl.ds(r0, RPT)])
    pltpu.sync_copy(zcnt_hbm.at[pl.ds(r0, RPT)], scnt.at[pl.ds(r0, RPT)])
    pltpu.sync_copy(ones_hbm, ones_v)
    plsc.subcore_barrier()

    def body(k, carry):
        bb = lax.rem(k, NBUF)
        nb = lax.rem(k + 2, NBUF)
        wait_gather(bb)

        @pl.when(k >= 1)
        def _():
            wait_scatters(nb)

        @pl.when(k + 2 < NCHUNK)
        def _():
            issue_gather(k + 2, nb)

        issue_scatters(bb)
        return carry

    lax.fori_loop(0, NCHUNK, body, 0)
    wait_scatters((NCHUNK - 1) % NBUF)
    plsc.subcore_barrier()

    pltpu.async_copy(ssum.at[pl.ds(r0, RPT)],
                     sums_out.at[pl.ds(cid * S + r0, RPT)], gsem.at[0])
    pltpu.async_copy(scnt.at[pl.ds(r0, RPT)],
                     cnts_out.at[pl.ds(cid * S + r0, RPT)], gsem.at[1])
    pltpu.make_async_copy(ssum.at[pl.ds(r0, RPT)],
                          sums_out.at[pl.ds(cid * S + r0, RPT)], gsem.at[0]).wait()
    pltpu.make_async_copy(scnt.at[pl.ds(r0, RPT)],
                          cnts_out.at[pl.ds(cid * S + r0, RPT)], gsem.at[1]).wait()


_sc_segment_sum = functools.partial(
    pl.kernel,
    out_type=[
        jax.ShapeDtypeStruct((NC * S, D), jnp.float32),
        jax.ShapeDtypeStruct((NC * S, CW), jnp.float32),
    ],
    mesh=plsc.VectorSubcoreMesh(core_axis_name="c", subcore_axis_name="s"),
    compiler_params=pltpu.CompilerParams(use_tc_tiling_on_sc=False),
    scratch_types=[
        pltpu.VMEM((NBUF * F, D), jnp.float32),
        pltpu.VMEM((NBUF, F), jnp.int32),
        pltpu.VMEM((F, CW), jnp.float32),
        pltpu.VMEM_SHARED((S, D), jnp.float32),
        pltpu.VMEM_SHARED((S, CW), jnp.float32),
        pltpu.SemaphoreType.DMA((NBUF,)),
        pltpu.SemaphoreType.DMA((NBUF,)),
    ],
)(_sc_body)


_BS = 1000


def _combine_body(s_ref, c_ref, o_ref):
    s = s_ref[0] + s_ref[1]
    c = c_ref[0, :, 0:1] + c_ref[1, :, 0:1]
    o_ref[...] = s / (c + 1e-8)


_combine = pl.pallas_call(
    _combine_body,
    grid=(S // _BS,),
    in_specs=[
        pl.BlockSpec((2, _BS, D), lambda i: (0, i, 0)),
        pl.BlockSpec((2, _BS, CW), lambda i: (0, i, 0)),
    ],
    out_specs=pl.BlockSpec((_BS, D), lambda i: (i, 0)),
    out_shape=jax.ShapeDtypeStruct((S, D), jnp.float32),
)


_ZSUM = np.zeros((S, D), np.float32)
_ZCNT = np.zeros((S, CW), np.float32)
_ONES = np.ones((F, CW), np.float32)


def kernel(frame_features, segment_ids, num_segments):
    ids2d = segment_ids.astype(jnp.int32).reshape(N // F, F)
    sums, cnts = _sc_segment_sum(frame_features, ids2d, _ZSUM, _ZCNT, _ONES)
    return _combine(sums.reshape(NC, S, D), cnts.reshape(NC, S, CW))
